# baseline (speedup 1.0000x reference)
import jax
import jax.numpy as jnp
from jax.experimental import pallas as pl
from jax.experimental.pallas import tpu as pltpu

T = 2048
H = 1024
F = 2048
E = 8
K = 2

NEG_INF = float("-inf")


def _router_kernel(x_ref, gwt_ref, w_ref):
    logits = jnp.dot(x_ref[...], gwt_ref[...], preferred_element_type=jnp.float32)
    m = jnp.max(logits, axis=-1, keepdims=True)
    p = jnp.exp(logits - m)
    p = p / jnp.sum(p, axis=-1, keepdims=True)
    idx = jax.lax.broadcasted_iota(jnp.int32, (T, E), 1)
    m1 = jnp.max(p, axis=-1, keepdims=True)
    i1 = jnp.min(jnp.where(p == m1, idx, E), axis=-1, keepdims=True)
    one1 = idx == i1
    p2 = jnp.where(one1, NEG_INF, p)
    m2 = jnp.max(p2, axis=-1, keepdims=True)
    i2 = jnp.min(jnp.where(p2 == m2, idx, E), axis=-1, keepdims=True)
    one2 = idx == i2
    denom = m1 + m2
    w_ref[...] = jnp.where(one1, m1 / denom, 0.0) + jnp.where(one2, m2 / denom, 0.0)


def _moe_kernel(x_ref, w1_ref, w2_ref, wfull_ref, out_ref):
    e = pl.program_id(1)
    x16 = x_ref[...].astype(jnp.bfloat16)
    h = jnp.dot(x16, w1_ref[0].astype(jnp.bfloat16),
                preferred_element_type=jnp.float32)
    h = h * jax.nn.sigmoid(h)
    y = jnp.dot(h.astype(jnp.bfloat16), w2_ref[0].astype(jnp.bfloat16),
                preferred_element_type=jnp.float32)
    ids = jax.lax.broadcasted_iota(jnp.int32, (1, E), 1)
    wcol = jnp.sum(wfull_ref[...] * (ids == e).astype(jnp.float32), axis=1,
                   keepdims=True)
    contrib = wcol * y

    @pl.when(e == 0)
    def _():
        out_ref[...] = contrib

    @pl.when(e > 0)
    def _():
        out_ref[...] += contrib


def kernel(hidden_states, gate_w, experts_w1, experts_w2):
    wfull = pl.pallas_call(
        _router_kernel,
        out_shape=jax.ShapeDtypeStruct((T, E), jnp.float32),
    )(hidden_states, gate_w.T)

    BT = 512
    NT = T // BT
    out = pl.pallas_call(
        _moe_kernel,
        grid=(NT, E),
        in_specs=[
            pl.BlockSpec((BT, H), lambda t, e: (t, 0)),
            pl.BlockSpec((1, H, F), lambda t, e: (e, 0, 0)),
            pl.BlockSpec((1, F, H), lambda t, e: (e, 0, 0)),
            pl.BlockSpec((BT, E), lambda t, e: (t, 0)),
        ],
        out_specs=pl.BlockSpec((BT, H), lambda t, e: (t, 0)),
        out_shape=jax.ShapeDtypeStruct((T, H), jnp.float32),
    )(hidden_states, experts_w1, experts_w2, wfull)
    return out


# speedup vs baseline: 1.0153x; 1.0153x over previous
import jax
import jax.numpy as jnp
from jax.experimental import pallas as pl
from jax.experimental.pallas import tpu as pltpu

T = 2048
H = 1024
F = 2048
E = 8
K = 2

NEG_INF = float("-inf")


def _router_kernel(x_ref, gwt_ref, w_ref):
    logits = jnp.dot(x_ref[...], gwt_ref[...], preferred_element_type=jnp.float32)
    m = jnp.max(logits, axis=-1, keepdims=True)
    p = jnp.exp(logits - m)
    p = p / jnp.sum(p, axis=-1, keepdims=True)
    idx = jax.lax.broadcasted_iota(jnp.int32, (T, E), 1)
    m1 = jnp.max(p, axis=-1, keepdims=True)
    i1 = jnp.min(jnp.where(p == m1, idx, E), axis=-1, keepdims=True)
    one1 = idx == i1
    p2 = jnp.where(one1, NEG_INF, p)
    m2 = jnp.max(p2, axis=-1, keepdims=True)
    i2 = jnp.min(jnp.where(p2 == m2, idx, E), axis=-1, keepdims=True)
    one2 = idx == i2
    denom = m1 + m2
    w_ref[...] = jnp.where(one1, m1 / denom, 0.0) + jnp.where(one2, m2 / denom, 0.0)


BT = 512
NT = T // BT


def _moe_kernel(x_ref, w1_ref, w2_ref, wfull_ref, out_ref):
    e = pl.program_id(0)
    t = pl.program_id(1)
    x16 = x_ref[...].astype(jnp.bfloat16)
    h = jnp.dot(x16, w1_ref[0].astype(jnp.bfloat16),
                preferred_element_type=jnp.float32)
    h = h * jax.nn.sigmoid(h)
    y = jnp.dot(h.astype(jnp.bfloat16), w2_ref[0].astype(jnp.bfloat16),
                preferred_element_type=jnp.float32)
    ids = jax.lax.broadcasted_iota(jnp.int32, (1, E), 1)
    wcol = jnp.sum(wfull_ref[...] * (ids == e).astype(jnp.float32), axis=1,
                   keepdims=True)
    contrib = wcol * y
    rows = pl.ds(t * BT, BT)

    @pl.when(e == 0)
    def _():
        out_ref[rows, :] = contrib

    @pl.when(e > 0)
    def _():
        out_ref[rows, :] += contrib


def kernel(hidden_states, gate_w, experts_w1, experts_w2):
    wfull = pl.pallas_call(
        _router_kernel,
        out_shape=jax.ShapeDtypeStruct((T, E), jnp.float32),
    )(hidden_states, gate_w.T)

    out = pl.pallas_call(
        _moe_kernel,
        grid=(E, NT),
        in_specs=[
            pl.BlockSpec((BT, H), lambda e, t: (t, 0)),
            pl.BlockSpec((1, H, F), lambda e, t: (e, 0, 0)),
            pl.BlockSpec((1, F, H), lambda e, t: (e, 0, 0)),
            pl.BlockSpec((BT, E), lambda e, t: (t, 0)),
        ],
        out_specs=pl.BlockSpec((T, H), lambda e, t: (0, 0)),
        out_shape=jax.ShapeDtypeStruct((T, H), jnp.float32),
    )(hidden_states, experts_w1, experts_w2, wfull)
    return out


# speedup vs baseline: 1.1404x; 1.1232x over previous
import functools

import jax
import jax.numpy as jnp
from jax import lax
from jax.experimental import pallas as pl
from jax.experimental.pallas import tpu as pltpu
from jax.experimental.pallas import tpu_sc as plsc

T = 2048
H = 1024
F = 2048
E = 8
K = 2

BT = 128
NBUF = K * T + E * BT
NTILES = NBUF // BT

NC = 2
NS = 16
NW = NC * NS
CHUNK = T // NW
CC = 32

NEG_INF = float("-inf")


def _router_kernel(x_ref, gw_ref, pos0_ref, pos1_ref, w_ref, te_ref):
    lg = lax.dot_general(gw_ref[...], x_ref[...], (((1,), (1,)), ((), ())),
                         preferred_element_type=jnp.float32)
    mx = jnp.max(lg, axis=0, keepdims=True)
    p = jnp.exp(lg - mx)
    iota0 = lax.broadcasted_iota(jnp.int32, (E, T), 0)
    m1 = jnp.max(p, axis=0, keepdims=True)
    i1 = jnp.min(jnp.where(p == m1, iota0, E), axis=0, keepdims=True)
    oh1 = iota0 == i1
    pm = jnp.where(oh1, NEG_INF, p)
    m2 = jnp.max(pm, axis=0, keepdims=True)
    i2 = jnp.min(jnp.where(pm == m2, iota0, E), axis=0, keepdims=True)
    oh2 = iota0 == i2
    denom = m1 + m2
    w_ref[...] = jnp.concatenate([m1 / denom, m2 / denom], axis=0)

    m0r = oh1.astype(jnp.float32).reshape(E, 16, 128).reshape(128, 128)
    m1r = oh2.astype(jnp.float32).reshape(E, 16, 128).reshape(128, 128)
    ri = lax.broadcasted_iota(jnp.int32, (128, 128), 0)
    ci = lax.broadcasted_iota(jnp.int32, (128, 128), 1)
    f32 = jnp.float32
    ustrict = (ri < ci).astype(f32)
    ones = jnp.ones((128, 128), f32)
    same_e = (ri // 16) == (ci // 16)
    lblk = (same_e & ((ci % 16) < (ri % 16))).astype(f32)
    allblk = same_e.astype(f32)
    rep = (ci % 16) == 0
    exlt = (((ci // 16) < (ri // 16)) & rep).astype(f32)
    exle = (((ci // 16) <= (ri // 16)) & rep).astype(f32)

    def mm(a, b):
        return jnp.dot(a, b, preferred_element_type=f32)

    p0 = mm(m0r, ustrict)
    p1 = mm(m1r, ustrict)
    s0 = mm(m0r, ones)
    s1 = mm(m1r, ones)
    b0 = mm(lblk, s0)
    b1 = mm(lblk, s1)
    c0 = mm(allblk, s0)
    cnt = c0 + mm(allblk, s1)
    pc = jnp.floor((cnt + (BT - 1)) * (1.0 / BT)) * BT
    off = mm(exlt, pc)
    pos0m = off + p0 + b0
    pos1m = off + p1 + b1 + c0
    pos0_ref[...] = (pos0m * m0r).reshape(E, 16, 128).sum(axis=0).astype(jnp.int32)
    pos1_ref[...] = (pos1m * m1r).reshape(E, 16, 128).sum(axis=0).astype(jnp.int32)

    ends = mm(exle, pc)
    cmp = (ends <= ci.astype(f32) * BT).astype(f32)
    sel = rep.astype(f32)
    te = jnp.minimum(mm(sel, cmp), E - 1)
    te_ref[...] = te[0:8, :].astype(jnp.int32)


def _gemm_kernel(te_ref, xs_ref, gs_ref, w1_ref, w2_ref, y_ref):
    x16 = xs_ref[...].astype(jnp.bfloat16)
    h = jnp.dot(x16, w1_ref[0].astype(jnp.bfloat16),
                preferred_element_type=jnp.float32)
    h = h * jax.nn.sigmoid(h)
    y = jnp.dot(h.astype(jnp.bfloat16), w2_ref[0].astype(jnp.bfloat16),
                preferred_element_type=jnp.float32)
    g = gs_ref[...]
    y_ref[...] = y * jnp.concatenate([g] * (H // 128), axis=1)


def _sc_mesh():
    return plsc.VectorSubcoreMesh(core_axis_name="c", subcore_axis_name="s",
                                  num_cores=NC, num_subcores=NS)


def _dispatch_body(x_hbm, g0_hbm, g1_hbm, pos0_hbm, pos1_hbm, xs_hbm, gs_hbm,
                   buf, bufg0, bufg1, idx0, idx1, sem0, sem1, sem2, sem3):
    wid = lax.axis_index("s") * NC + lax.axis_index("c")
    base = wid * CHUNK
    pltpu.sync_copy(x_hbm.at[pl.ds(base, CHUNK)], buf)
    pltpu.sync_copy(g0_hbm.at[pl.ds(base, CHUNK)], bufg0)
    pltpu.sync_copy(g1_hbm.at[pl.ds(base, CHUNK)], bufg1)
    pltpu.sync_copy(pos0_hbm.at[pl.ds(base, CHUNK)], idx0)
    pltpu.sync_copy(pos1_hbm.at[pl.ds(base, CHUNK)], idx1)
    cp0 = pltpu.async_copy(buf, xs_hbm.at[idx0], sem0)
    cp1 = pltpu.async_copy(buf, xs_hbm.at[idx1], sem1)
    cp2 = pltpu.async_copy(bufg0, gs_hbm.at[idx0], sem2)
    cp3 = pltpu.async_copy(bufg1, gs_hbm.at[idx1], sem3)
    cp0.wait()
    cp1.wait()
    cp2.wait()
    cp3.wait()


@functools.cache
def _make_dispatch():
    return pl.kernel(
        _dispatch_body,
        out_type=(jax.ShapeDtypeStruct((NBUF, H), jnp.float32),
                  jax.ShapeDtypeStruct((NBUF, 128), jnp.float32)),
        mesh=_sc_mesh(),
        scratch_types=[
            pltpu.VMEM((CHUNK, H), jnp.float32),
            pltpu.VMEM((CHUNK, 128), jnp.float32),
            pltpu.VMEM((CHUNK, 128), jnp.float32),
            pltpu.VMEM((CHUNK,), jnp.int32),
            pltpu.VMEM((CHUNK,), jnp.int32),
            pltpu.SemaphoreType.DMA,
            pltpu.SemaphoreType.DMA,
            pltpu.SemaphoreType.DMA,
            pltpu.SemaphoreType.DMA,
        ],
    )


def _dispatch(x, g0, g1, pos0, pos1):
    return _make_dispatch()(x, g0, g1, pos0, pos1)


def _combine_body(y_hbm, pos0_hbm, pos1_hbm, out_hbm,
                  bufa, bufb, idx0, idx1, sema, semb):
    wid = lax.axis_index("s") * NC + lax.axis_index("c")
    for hh in range(CHUNK // CC):
        base = wid * CHUNK + hh * CC
        pltpu.sync_copy(pos0_hbm.at[pl.ds(base, CC)], idx0)
        pltpu.sync_copy(pos1_hbm.at[pl.ds(base, CC)], idx1)
        cpa = pltpu.async_copy(y_hbm.at[idx0], bufa, sema)
        cpb = pltpu.async_copy(y_hbm.at[idx1], bufb, semb)
        cpa.wait()
        cpb.wait()

        def vec(i, c3):
            j = i // (H // 16)
            sl = pl.ds((i % (H // 16)) * 16, 16)
            bufa[j, sl] = bufa[j, sl] + bufb[j, sl]
            return c3

        lax.fori_loop(0, CC * (H // 16), vec, 0)
        pltpu.sync_copy(bufa, out_hbm.at[pl.ds(base, CC)])


@functools.cache
def _make_combine():
    return pl.kernel(
        _combine_body,
        out_type=jax.ShapeDtypeStruct((T, H), jnp.float32),
        mesh=_sc_mesh(),
        scratch_types=[
            pltpu.VMEM((CC, H), jnp.float32),
            pltpu.VMEM((CC, H), jnp.float32),
            pltpu.VMEM((CC,), jnp.int32),
            pltpu.VMEM((CC,), jnp.int32),
            pltpu.SemaphoreType.DMA,
            pltpu.SemaphoreType.DMA,
        ],
    )


def _combine(y, pos0, pos1):
    return _make_combine()(y, pos0, pos1)


def kernel(hidden_states, gate_w, experts_w1, experts_w2):
    pos0g, pos1g, wg, teg = pl.pallas_call(
        _router_kernel,
        out_shape=(
            jax.ShapeDtypeStruct((16, 128), jnp.int32),
            jax.ShapeDtypeStruct((16, 128), jnp.int32),
            jax.ShapeDtypeStruct((2, T), jnp.float32),
            jax.ShapeDtypeStruct((8, 128), jnp.int32),
        ),
    )(hidden_states, gate_w)
    pos0 = pos0g.reshape(T)
    pos1 = pos1g.reshape(T)
    g0 = jnp.broadcast_to(wg[0][:, None], (T, 128))
    g1 = jnp.broadcast_to(wg[1][:, None], (T, 128))
    te = teg[0, :NTILES]

    x_sorted, g_sorted = _dispatch(hidden_states, g0, g1, pos0, pos1)

    y = pl.pallas_call(
        _gemm_kernel,
        grid_spec=pltpu.PrefetchScalarGridSpec(
            num_scalar_prefetch=1,
            grid=(NTILES,),
            in_specs=[
                pl.BlockSpec((BT, H), lambda j, te_s: (j, 0)),
                pl.BlockSpec((BT, 128), lambda j, te_s: (j, 0)),
                pl.BlockSpec((1, H, F), lambda j, te_s: (te_s[j], 0, 0)),
                pl.BlockSpec((1, F, H), lambda j, te_s: (te_s[j], 0, 0)),
            ],
            out_specs=pl.BlockSpec((BT, H), lambda j, te_s: (j, 0)),
        ),
        out_shape=jax.ShapeDtypeStruct((NBUF, H), jnp.float32),
    )(te, x_sorted, g_sorted, experts_w1, experts_w2)

    return _combine(y, pos0, pos1)


# speedup vs baseline: 1.1463x; 1.0052x over previous
import functools

import jax
import jax.numpy as jnp
from jax import lax
from jax.experimental import pallas as pl
from jax.experimental.pallas import tpu as pltpu
from jax.experimental.pallas import tpu_sc as plsc

T = 2048
H = 1024
F = 2048
E = 8
K = 2

BT = 128
NBUF = K * T + E * BT
NTILES = NBUF // BT

NC = 2
NS = 16
NW = NC * NS
CHUNK = T // NW
CC = 32

NEG_INF = float("-inf")


def _router_kernel(x_ref, gw_ref, pos0_ref, pos1_ref, g0_ref, g1_ref, te_ref):
    lg = lax.dot_general(gw_ref[...], x_ref[...], (((1,), (1,)), ((), ())),
                         preferred_element_type=jnp.float32)
    mx = jnp.max(lg, axis=0, keepdims=True)
    p = jnp.exp(lg - mx)
    iota0 = lax.broadcasted_iota(jnp.int32, (E, T), 0)
    m1 = jnp.max(p, axis=0, keepdims=True)
    i1 = jnp.min(jnp.where(p == m1, iota0, E), axis=0, keepdims=True)
    oh1 = iota0 == i1
    pm = jnp.where(oh1, NEG_INF, p)
    m2 = jnp.max(pm, axis=0, keepdims=True)
    i2 = jnp.min(jnp.where(pm == m2, iota0, E), axis=0, keepdims=True)
    oh2 = iota0 == i2
    denom = m1 + m2
    ones_row = jnp.ones((1, 128), jnp.float32)
    dn = (((0,), (0,)), ((), ()))
    g0_ref[...] = lax.dot_general(m1 / denom, ones_row, dn,
                                  precision=lax.Precision.HIGHEST,
                                  preferred_element_type=jnp.float32)
    g1_ref[...] = lax.dot_general(m2 / denom, ones_row, dn,
                                  precision=lax.Precision.HIGHEST,
                                  preferred_element_type=jnp.float32)

    m0r = oh1.astype(jnp.float32).reshape(E, 16, 128).reshape(128, 128)
    m1r = oh2.astype(jnp.float32).reshape(E, 16, 128).reshape(128, 128)
    ri = lax.broadcasted_iota(jnp.int32, (128, 128), 0)
    ci = lax.broadcasted_iota(jnp.int32, (128, 128), 1)
    f32 = jnp.float32
    ustrict = (ri < ci).astype(f32)
    ones = jnp.ones((128, 128), f32)
    same_e = (ri // 16) == (ci // 16)
    lblk = (same_e & ((ci % 16) < (ri % 16))).astype(f32)
    allblk = same_e.astype(f32)
    rep = (ci % 16) == 0
    exlt = (((ci // 16) < (ri // 16)) & rep).astype(f32)
    exle = (((ci // 16) <= (ri // 16)) & rep).astype(f32)

    def mm(a, b):
        return jnp.dot(a, b, preferred_element_type=f32)

    p0 = mm(m0r, ustrict)
    p1 = mm(m1r, ustrict)
    s0 = mm(m0r, ones)
    s1 = mm(m1r, ones)
    b0 = mm(lblk, s0)
    b1 = mm(lblk, s1)
    c0 = mm(allblk, s0)
    cnt = c0 + mm(allblk, s1)
    pc = jnp.floor((cnt + (BT - 1)) * (1.0 / BT)) * BT
    off = mm(exlt, pc)
    pos0m = off + p0 + b0
    pos1m = off + p1 + b1 + c0
    pos0_ref[...] = (pos0m * m0r).reshape(E, 16, 128).sum(axis=0).astype(jnp.int32)
    pos1_ref[...] = (pos1m * m1r).reshape(E, 16, 128).sum(axis=0).astype(jnp.int32)

    ends = mm(exle, pc)
    cmp = (ends <= ci.astype(f32) * BT).astype(f32)
    sel = rep.astype(f32)
    te = jnp.minimum(mm(sel, cmp), E - 1)
    te_ref[...] = te[0:8, :].astype(jnp.int32)


def _gemm_kernel(te_ref, xs_ref, gs_ref, w1_ref, w2_ref, y_ref):
    x16 = xs_ref[...].astype(jnp.bfloat16)
    h = jnp.dot(x16, w1_ref[0].astype(jnp.bfloat16),
                preferred_element_type=jnp.float32)
    h = h * jax.nn.sigmoid(h)
    y = jnp.dot(h.astype(jnp.bfloat16), w2_ref[0].astype(jnp.bfloat16),
                preferred_element_type=jnp.float32)
    g = gs_ref[...]
    y_ref[...] = y * jnp.concatenate([g] * (H // 128), axis=1)


def _sc_mesh():
    return plsc.VectorSubcoreMesh(core_axis_name="c", subcore_axis_name="s",
                                  num_cores=NC, num_subcores=NS)


def _dispatch_body(x_hbm, g0_hbm, g1_hbm, pos0_hbm, pos1_hbm, xs_hbm, gs_hbm,
                   buf, bufg0, bufg1, idx0, idx1, sem0, sem1, sem2, sem3):
    wid = lax.axis_index("s") * NC + lax.axis_index("c")
    base = wid * CHUNK
    pltpu.sync_copy(x_hbm.at[pl.ds(base, CHUNK)], buf)
    pltpu.sync_copy(g0_hbm.at[pl.ds(base, CHUNK)], bufg0)
    pltpu.sync_copy(g1_hbm.at[pl.ds(base, CHUNK)], bufg1)
    pltpu.sync_copy(pos0_hbm.at[pl.ds(base, CHUNK)], idx0)
    pltpu.sync_copy(pos1_hbm.at[pl.ds(base, CHUNK)], idx1)
    cp0 = pltpu.async_copy(buf, xs_hbm.at[idx0], sem0)
    cp1 = pltpu.async_copy(buf, xs_hbm.at[idx1], sem1)
    cp2 = pltpu.async_copy(bufg0, gs_hbm.at[idx0], sem2)
    cp3 = pltpu.async_copy(bufg1, gs_hbm.at[idx1], sem3)
    cp0.wait()
    cp1.wait()
    cp2.wait()
    cp3.wait()


@functools.cache
def _make_dispatch():
    return pl.kernel(
        _dispatch_body,
        out_type=(jax.ShapeDtypeStruct((NBUF, H), jnp.float32),
                  jax.ShapeDtypeStruct((NBUF, 128), jnp.float32)),
        mesh=_sc_mesh(),
        scratch_types=[
            pltpu.VMEM((CHUNK, H), jnp.float32),
            pltpu.VMEM((CHUNK, 128), jnp.float32),
            pltpu.VMEM((CHUNK, 128), jnp.float32),
            pltpu.VMEM((CHUNK,), jnp.int32),
            pltpu.VMEM((CHUNK,), jnp.int32),
            pltpu.SemaphoreType.DMA,
            pltpu.SemaphoreType.DMA,
            pltpu.SemaphoreType.DMA,
            pltpu.SemaphoreType.DMA,
        ],
    )


def _dispatch(x, g0, g1, pos0, pos1):
    return _make_dispatch()(x, g0, g1, pos0, pos1)


def _combine_body(y_hbm, pos0_hbm, pos1_hbm, out_hbm,
                  bufa, bufb, idx0, idx1, sema, semb):
    wid = lax.axis_index("c") * NS + lax.axis_index("s")
    for hh in range(CHUNK // CC):
        base = wid * CHUNK + hh * CC
        pltpu.sync_copy(pos0_hbm.at[pl.ds(base, CC)], idx0)
        pltpu.sync_copy(pos1_hbm.at[pl.ds(base, CC)], idx1)
        cpa = pltpu.async_copy(y_hbm.at[idx0], bufa, sema)
        cpb = pltpu.async_copy(y_hbm.at[idx1], bufb, semb)
        cpa.wait()
        cpb.wait()

        def vec(i, c3):
            j = i // (H // 16)
            sl = pl.ds((i % (H // 16)) * 16, 16)
            bufa[j, sl] = bufa[j, sl] + bufb[j, sl]
            return c3

        lax.fori_loop(0, CC * (H // 16), vec, 0)
        pltpu.sync_copy(bufa, out_hbm.at[pl.ds(base, CC)])


@functools.cache
def _make_combine():
    return pl.kernel(
        _combine_body,
        out_type=jax.ShapeDtypeStruct((T, H), jnp.float32),
        mesh=_sc_mesh(),
        scratch_types=[
            pltpu.VMEM((CC, H), jnp.float32),
            pltpu.VMEM((CC, H), jnp.float32),
            pltpu.VMEM((CC,), jnp.int32),
            pltpu.VMEM((CC,), jnp.int32),
            pltpu.SemaphoreType.DMA,
            pltpu.SemaphoreType.DMA,
        ],
    )


def _combine(y, pos0, pos1):
    return _make_combine()(y, pos0, pos1)


def kernel(hidden_states, gate_w, experts_w1, experts_w2):
    pos0g, pos1g, g0, g1, teg = pl.pallas_call(
        _router_kernel,
        out_shape=(
            jax.ShapeDtypeStruct((16, 128), jnp.int32),
            jax.ShapeDtypeStruct((16, 128), jnp.int32),
            jax.ShapeDtypeStruct((T, 128), jnp.float32),
            jax.ShapeDtypeStruct((T, 128), jnp.float32),
            jax.ShapeDtypeStruct((8, 128), jnp.int32),
        ),
    )(hidden_states, gate_w)
    pos0 = pos0g.reshape(T)
    pos1 = pos1g.reshape(T)

    x_sorted, g_sorted = _dispatch(hidden_states, g0, g1, pos0, pos1)

    y = pl.pallas_call(
        _gemm_kernel,
        grid_spec=pltpu.PrefetchScalarGridSpec(
            num_scalar_prefetch=1,
            grid=(NTILES,),
            in_specs=[
                pl.BlockSpec((BT, H), lambda j, te_s: (j, 0)),
                pl.BlockSpec((BT, 128), lambda j, te_s: (j, 0)),
                pl.BlockSpec((1, H, F), lambda j, te_s: (te_s[0, j], 0, 0)),
                pl.BlockSpec((1, F, H), lambda j, te_s: (te_s[0, j], 0, 0)),
            ],
            out_specs=pl.BlockSpec((BT, H), lambda j, te_s: (j, 0)),
        ),
        out_shape=jax.ShapeDtypeStruct((NBUF, H), jnp.float32),
    )(teg, x_sorted, g_sorted, experts_w1, experts_w2)

    return _combine(y, pos0, pos1)


# speedup vs baseline: 1.2246x; 1.0683x over previous
import functools

import jax
import jax.numpy as jnp
from jax import lax
from jax.experimental import pallas as pl
from jax.experimental.pallas import tpu as pltpu
from jax.experimental.pallas import tpu_sc as plsc

T = 2048
H = 1024
F = 2048
E = 8
K = 2

BT = 128
NBUF = K * T + E * BT
NTILES = NBUF // BT

NC = 2
NS = 16
NW = NC * NS
CHUNK = T // NW
CC = 32

NEG_INF = float("-inf")


def _router_kernel(x_ref, gw_ref, pos0_ref, pos1_ref, g0_ref, g1_ref, te_ref):
    lg = lax.dot_general(gw_ref[...], x_ref[...], (((1,), (1,)), ((), ())),
                         preferred_element_type=jnp.float32)
    mx = jnp.max(lg, axis=0, keepdims=True)
    p = jnp.exp(lg - mx)
    iota0 = lax.broadcasted_iota(jnp.int32, (E, T), 0)
    m1 = jnp.max(p, axis=0, keepdims=True)
    i1 = jnp.min(jnp.where(p == m1, iota0, E), axis=0, keepdims=True)
    oh1 = iota0 == i1
    pm = jnp.where(oh1, NEG_INF, p)
    m2 = jnp.max(pm, axis=0, keepdims=True)
    i2 = jnp.min(jnp.where(pm == m2, iota0, E), axis=0, keepdims=True)
    oh2 = iota0 == i2
    denom = m1 + m2
    ones_row = jnp.ones((1, 128), jnp.float32)
    dn = (((0,), (0,)), ((), ()))
    g0_ref[...] = lax.dot_general(m1 / denom, ones_row, dn,
                                  precision=lax.Precision.HIGHEST,
                                  preferred_element_type=jnp.float32)
    g1_ref[...] = lax.dot_general(m2 / denom, ones_row, dn,
                                  precision=lax.Precision.HIGHEST,
                                  preferred_element_type=jnp.float32)

    m0r = oh1.astype(jnp.float32).reshape(E, 16, 128).reshape(128, 128)
    m1r = oh2.astype(jnp.float32).reshape(E, 16, 128).reshape(128, 128)
    ri = lax.broadcasted_iota(jnp.int32, (128, 128), 0)
    ci = lax.broadcasted_iota(jnp.int32, (128, 128), 1)
    f32 = jnp.float32
    ustrict = (ri < ci).astype(f32)
    ones = jnp.ones((128, 128), f32)
    same_e = (ri // 16) == (ci // 16)
    lblk = (same_e & ((ci % 16) < (ri % 16))).astype(f32)
    allblk = same_e.astype(f32)
    rep = (ci % 16) == 0
    exlt = (((ci // 16) < (ri // 16)) & rep).astype(f32)
    exle = (((ci // 16) <= (ri // 16)) & rep).astype(f32)

    def mm(a, b):
        return jnp.dot(a, b, preferred_element_type=f32)

    p0 = mm(m0r, ustrict)
    p1 = mm(m1r, ustrict)
    s0 = mm(m0r, ones)
    s1 = mm(m1r, ones)
    b0 = mm(lblk, s0)
    b1 = mm(lblk, s1)
    c0 = mm(allblk, s0)
    cnt = c0 + mm(allblk, s1)
    pc = jnp.floor((cnt + (BT - 1)) * (1.0 / BT)) * BT
    off = mm(exlt, pc)
    pos0m = off + p0 + b0
    pos1m = off + p1 + b1 + c0
    pos0_ref[...] = (pos0m * m0r).reshape(E, 16, 128).sum(axis=0).astype(jnp.int32)
    pos1_ref[...] = (pos1m * m1r).reshape(E, 16, 128).sum(axis=0).astype(jnp.int32)

    ends = mm(exle, pc)
    cmp = (ends <= ci.astype(f32) * BT).astype(f32)
    sel = rep.astype(f32)
    te = jnp.minimum(mm(sel, cmp), E - 1)
    te_ref[...] = te[0:8, :].astype(jnp.int32)


def _gemm_kernel(te_ref, xs_ref, gs_ref, w1_ref, w2_ref, y_ref):
    x16 = xs_ref[...].astype(jnp.bfloat16)
    h = jnp.dot(x16, w1_ref[0].astype(jnp.bfloat16),
                preferred_element_type=jnp.float32)
    h = h * jax.nn.sigmoid(h)
    y = jnp.dot(h.astype(jnp.bfloat16), w2_ref[0].astype(jnp.bfloat16),
                preferred_element_type=jnp.float32)
    g = gs_ref[...]
    y_ref[...] = y * jnp.concatenate([g] * (H // 128), axis=1)


def _sc_mesh():
    return plsc.VectorSubcoreMesh(core_axis_name="c", subcore_axis_name="s",
                                  num_cores=NC, num_subcores=NS)


def _dispatch_body(x_hbm, g0_hbm, g1_hbm, pos0_hbm, pos1_hbm, xs_hbm, gs_hbm,
                   buf, bufg0, bufg1, idx0, idx1, sem0, sem1, sem2, sem3):
    wid = lax.axis_index("s") * NC + lax.axis_index("c")
    base = wid * CHUNK
    pltpu.sync_copy(x_hbm.at[pl.ds(base, CHUNK)], buf)
    pltpu.sync_copy(g0_hbm.at[pl.ds(base, CHUNK)], bufg0)
    pltpu.sync_copy(g1_hbm.at[pl.ds(base, CHUNK)], bufg1)
    pltpu.sync_copy(pos0_hbm.at[pl.ds(base, CHUNK)], idx0)
    pltpu.sync_copy(pos1_hbm.at[pl.ds(base, CHUNK)], idx1)
    cp0 = pltpu.async_copy(buf, xs_hbm.at[idx0], sem0)
    cp1 = pltpu.async_copy(buf, xs_hbm.at[idx1], sem1)
    cp2 = pltpu.async_copy(bufg0, gs_hbm.at[idx0], sem2)
    cp3 = pltpu.async_copy(bufg1, gs_hbm.at[idx1], sem3)
    cp0.wait()
    cp1.wait()
    cp2.wait()
    cp3.wait()


@functools.cache
def _make_dispatch():
    return pl.kernel(
        _dispatch_body,
        out_type=(jax.ShapeDtypeStruct((NBUF, H), jnp.float32),
                  jax.ShapeDtypeStruct((NBUF, 128), jnp.float32)),
        mesh=_sc_mesh(),
        scratch_types=[
            pltpu.VMEM((CHUNK, H), jnp.float32),
            pltpu.VMEM((CHUNK, 128), jnp.float32),
            pltpu.VMEM((CHUNK, 128), jnp.float32),
            pltpu.VMEM((CHUNK,), jnp.int32),
            pltpu.VMEM((CHUNK,), jnp.int32),
            pltpu.SemaphoreType.DMA,
            pltpu.SemaphoreType.DMA,
            pltpu.SemaphoreType.DMA,
            pltpu.SemaphoreType.DMA,
        ],
    )


def _dispatch(x, g0, g1, pos0, pos1):
    return _make_dispatch()(x, g0, g1, pos0, pos1)


def _combine_body(y_hbm, pos0_hbm, pos1_hbm, out_hbm,
                  bufa, bufb, idx0, idx1, sema, semb):
    wid = lax.axis_index("c") * NS + lax.axis_index("s")
    for hh in range(CHUNK // CC):
        base = wid * CHUNK + hh * CC
        pltpu.sync_copy(pos0_hbm.at[pl.ds(base, CC)], idx0)
        pltpu.sync_copy(pos1_hbm.at[pl.ds(base, CC)], idx1)
        cpa = pltpu.async_copy(y_hbm.at[idx0], bufa, sema)
        cpb = pltpu.async_copy(y_hbm.at[idx1], bufb, semb)
        cpa.wait()
        cpb.wait()

        def tokrow(j, c3):
            for v in range(0, H // 16, 8):
                for u in range(8):
                    sl = pl.ds((v + u) * 16, 16)
                    bufa[j, sl] = bufa[j, sl] + bufb[j, sl]
            return c3

        lax.fori_loop(0, CC, tokrow, 0)
        pltpu.sync_copy(bufa, out_hbm.at[pl.ds(base, CC)])


@functools.cache
def _make_combine():
    return pl.kernel(
        _combine_body,
        out_type=jax.ShapeDtypeStruct((T, H), jnp.float32),
        mesh=_sc_mesh(),
        scratch_types=[
            pltpu.VMEM((CC, H), jnp.float32),
            pltpu.VMEM((CC, H), jnp.float32),
            pltpu.VMEM((CC,), jnp.int32),
            pltpu.VMEM((CC,), jnp.int32),
            pltpu.SemaphoreType.DMA,
            pltpu.SemaphoreType.DMA,
        ],
    )


def _combine(y, pos0, pos1):
    return _make_combine()(y, pos0, pos1)


def kernel(hidden_states, gate_w, experts_w1, experts_w2):
    pos0g, pos1g, g0, g1, teg = pl.pallas_call(
        _router_kernel,
        out_shape=(
            jax.ShapeDtypeStruct((16, 128), jnp.int32),
            jax.ShapeDtypeStruct((16, 128), jnp.int32),
            jax.ShapeDtypeStruct((T, 128), jnp.float32),
            jax.ShapeDtypeStruct((T, 128), jnp.float32),
            jax.ShapeDtypeStruct((8, 128), jnp.int32),
        ),
    )(hidden_states, gate_w)
    pos0 = pos0g.reshape(T)
    pos1 = pos1g.reshape(T)

    x_sorted, g_sorted = _dispatch(hidden_states, g0, g1, pos0, pos1)

    y = pl.pallas_call(
        _gemm_kernel,
        grid_spec=pltpu.PrefetchScalarGridSpec(
            num_scalar_prefetch=1,
            grid=(NTILES,),
            in_specs=[
                pl.BlockSpec((BT, H), lambda j, te_s: (j, 0)),
                pl.BlockSpec((BT, 128), lambda j, te_s: (j, 0)),
                pl.BlockSpec((1, H, F), lambda j, te_s: (te_s[0, j], 0, 0)),
                pl.BlockSpec((1, F, H), lambda j, te_s: (te_s[0, j], 0, 0)),
            ],
            out_specs=pl.BlockSpec((BT, H), lambda j, te_s: (j, 0)),
        ),
        out_shape=jax.ShapeDtypeStruct((NBUF, H), jnp.float32),
    )(teg, x_sorted, g_sorted, experts_w1, experts_w2)

    return _combine(y, pos0, pos1)


# speedup vs baseline: 1.2686x; 1.0359x over previous
import functools

import jax
import jax.numpy as jnp
from jax import lax
from jax.experimental import pallas as pl
from jax.experimental.pallas import tpu as pltpu
from jax.experimental.pallas import tpu_sc as plsc

T = 2048
H = 1024
F = 2048
E = 8
K = 2

BT = 128
NBUF = K * T + E * BT
NTILES = NBUF // BT

NC = 2
NS = 16
NW = NC * NS
CHUNK = T // NW
CC = 32

NEG_INF = float("-inf")


def _router_kernel(x_ref, gw_ref, pos0_ref, pos1_ref, g0_ref, g1_ref, te_ref):
    lg = lax.dot_general(gw_ref[...], x_ref[...], (((1,), (1,)), ((), ())),
                         preferred_element_type=jnp.float32)
    mx = jnp.max(lg, axis=0, keepdims=True)
    p = jnp.exp(lg - mx)
    iota0 = lax.broadcasted_iota(jnp.int32, (E, T), 0)
    m1 = jnp.max(p, axis=0, keepdims=True)
    i1 = jnp.min(jnp.where(p == m1, iota0, E), axis=0, keepdims=True)
    oh1 = iota0 == i1
    pm = jnp.where(oh1, NEG_INF, p)
    m2 = jnp.max(pm, axis=0, keepdims=True)
    i2 = jnp.min(jnp.where(pm == m2, iota0, E), axis=0, keepdims=True)
    oh2 = iota0 == i2
    denom = m1 + m2
    ones_row = jnp.ones((1, 128), jnp.float32)
    dn = (((0,), (0,)), ((), ()))
    g0_ref[...] = lax.dot_general(m1 / denom, ones_row, dn,
                                  precision=lax.Precision.HIGHEST,
                                  preferred_element_type=jnp.float32)
    g1_ref[...] = lax.dot_general(m2 / denom, ones_row, dn,
                                  precision=lax.Precision.HIGHEST,
                                  preferred_element_type=jnp.float32)

    m0r = oh1.astype(jnp.float32).reshape(E, 16, 128).reshape(128, 128)
    m1r = oh2.astype(jnp.float32).reshape(E, 16, 128).reshape(128, 128)
    ri = lax.broadcasted_iota(jnp.int32, (128, 128), 0)
    ci = lax.broadcasted_iota(jnp.int32, (128, 128), 1)
    f32 = jnp.float32
    ustrict = (ri < ci).astype(f32)
    ones = jnp.ones((128, 128), f32)
    same_e = (ri // 16) == (ci // 16)
    lblk = (same_e & ((ci % 16) < (ri % 16))).astype(f32)
    allblk = same_e.astype(f32)
    rep = (ci % 16) == 0
    exlt = (((ci // 16) < (ri // 16)) & rep).astype(f32)
    exle = (((ci // 16) <= (ri // 16)) & rep).astype(f32)

    def mm(a, b):
        return jnp.dot(a, b, preferred_element_type=f32)

    p0 = mm(m0r, ustrict)
    p1 = mm(m1r, ustrict)
    s0 = mm(m0r, ones)
    s1 = mm(m1r, ones)
    b0 = mm(lblk, s0)
    b1 = mm(lblk, s1)
    c0 = mm(allblk, s0)
    cnt = c0 + mm(allblk, s1)
    pc = jnp.floor((cnt + (BT - 1)) * (1.0 / BT)) * BT
    off = mm(exlt, pc)
    pos0m = off + p0 + b0
    pos1m = off + p1 + b1 + c0
    pos0_ref[...] = (pos0m * m0r).reshape(E, 16, 128).sum(axis=0).astype(jnp.int32)
    pos1_ref[...] = (pos1m * m1r).reshape(E, 16, 128).sum(axis=0).astype(jnp.int32)

    ends = mm(exle, pc)
    cmp = (ends <= ci.astype(f32) * BT).astype(f32)
    sel = rep.astype(f32)
    te = jnp.minimum(mm(sel, cmp), E - 1)
    ntil = mm(sel, pc) * (1.0 / BT)
    te_ref[...] = jnp.concatenate([te[0:1], ntil[0:1], te[2:8]],
                                  axis=0).astype(jnp.int32)


def _gemm_kernel(te_ref, xs_ref, gs_ref, w1_ref, w2_ref, y_ref):
    @pl.when(pl.program_id(0) < te_ref[1, 0])
    def _():
        x16 = xs_ref[...].astype(jnp.bfloat16)
        h = jnp.dot(x16, w1_ref[0].astype(jnp.bfloat16),
                    preferred_element_type=jnp.float32)
        h = h * jax.nn.sigmoid(h)
        y = jnp.dot(h.astype(jnp.bfloat16), w2_ref[0].astype(jnp.bfloat16),
                    preferred_element_type=jnp.float32)
        g = gs_ref[...]
        y_ref[...] = y * jnp.concatenate([g] * (H // 128), axis=1)


def _sc_mesh():
    return plsc.VectorSubcoreMesh(core_axis_name="c", subcore_axis_name="s",
                                  num_cores=NC, num_subcores=NS)


def _dispatch_body(x_hbm, g0_hbm, g1_hbm, pos0_hbm, pos1_hbm, xs_hbm, gs_hbm,
                   buf, bufg0, bufg1, idx0, idx1, sem0, sem1, sem2, sem3):
    wid = lax.axis_index("s") * NC + lax.axis_index("c")
    base = wid * CHUNK
    pltpu.sync_copy(x_hbm.at[pl.ds(base, CHUNK)], buf)
    pltpu.sync_copy(g0_hbm.at[pl.ds(base, CHUNK)], bufg0)
    pltpu.sync_copy(g1_hbm.at[pl.ds(base, CHUNK)], bufg1)
    pltpu.sync_copy(pos0_hbm.at[pl.ds(base, CHUNK)], idx0)
    pltpu.sync_copy(pos1_hbm.at[pl.ds(base, CHUNK)], idx1)
    cp0 = pltpu.async_copy(buf, xs_hbm.at[idx0], sem0)
    cp1 = pltpu.async_copy(buf, xs_hbm.at[idx1], sem1)
    cp2 = pltpu.async_copy(bufg0, gs_hbm.at[idx0], sem2)
    cp3 = pltpu.async_copy(bufg1, gs_hbm.at[idx1], sem3)
    cp0.wait()
    cp1.wait()
    cp2.wait()
    cp3.wait()


@functools.cache
def _make_dispatch():
    return pl.kernel(
        _dispatch_body,
        out_type=(jax.ShapeDtypeStruct((NBUF, H), jnp.float32),
                  jax.ShapeDtypeStruct((NBUF, 128), jnp.float32)),
        mesh=_sc_mesh(),
        scratch_types=[
            pltpu.VMEM((CHUNK, H), jnp.float32),
            pltpu.VMEM((CHUNK, 128), jnp.float32),
            pltpu.VMEM((CHUNK, 128), jnp.float32),
            pltpu.VMEM((CHUNK,), jnp.int32),
            pltpu.VMEM((CHUNK,), jnp.int32),
            pltpu.SemaphoreType.DMA,
            pltpu.SemaphoreType.DMA,
            pltpu.SemaphoreType.DMA,
            pltpu.SemaphoreType.DMA,
        ],
    )


def _dispatch(x, g0, g1, pos0, pos1):
    return _make_dispatch()(x, g0, g1, pos0, pos1)


def _combine_body(y_hbm, pos0_hbm, pos1_hbm, out_hbm,
                  bufa, bufb, idx0, idx1, sema, semb):
    wid = lax.axis_index("c") * NS + lax.axis_index("s")
    for hh in range(CHUNK // CC):
        base = wid * CHUNK + hh * CC
        pltpu.sync_copy(pos0_hbm.at[pl.ds(base, CC)], idx0)
        pltpu.sync_copy(pos1_hbm.at[pl.ds(base, CC)], idx1)
        cpa = pltpu.async_copy(y_hbm.at[idx0], bufa, sema)
        cpb = pltpu.async_copy(y_hbm.at[idx1], bufb, semb)
        cpa.wait()
        cpb.wait()

        def tokrow(j, c3):
            for v in range(0, H // 16, 8):
                for u in range(8):
                    sl = pl.ds((v + u) * 16, 16)
                    bufa[j, sl] = bufa[j, sl] + bufb[j, sl]
            return c3

        lax.fori_loop(0, CC, tokrow, 0)
        pltpu.sync_copy(bufa, out_hbm.at[pl.ds(base, CC)])


@functools.cache
def _make_combine():
    return pl.kernel(
        _combine_body,
        out_type=jax.ShapeDtypeStruct((T, H), jnp.float32),
        mesh=_sc_mesh(),
        scratch_types=[
            pltpu.VMEM((CC, H), jnp.float32),
            pltpu.VMEM((CC, H), jnp.float32),
            pltpu.VMEM((CC,), jnp.int32),
            pltpu.VMEM((CC,), jnp.int32),
            pltpu.SemaphoreType.DMA,
            pltpu.SemaphoreType.DMA,
        ],
    )


def _combine(y, pos0, pos1):
    return _make_combine()(y, pos0, pos1)


def kernel(hidden_states, gate_w, experts_w1, experts_w2):
    pos0g, pos1g, g0, g1, teg = pl.pallas_call(
        _router_kernel,
        out_shape=(
            jax.ShapeDtypeStruct((16, 128), jnp.int32),
            jax.ShapeDtypeStruct((16, 128), jnp.int32),
            jax.ShapeDtypeStruct((T, 128), jnp.float32),
            jax.ShapeDtypeStruct((T, 128), jnp.float32),
            jax.ShapeDtypeStruct((8, 128), jnp.int32),
        ),
    )(hidden_states, gate_w)
    pos0 = pos0g.reshape(T)
    pos1 = pos1g.reshape(T)

    x_sorted, g_sorted = _dispatch(hidden_states, g0, g1, pos0, pos1)

    y = pl.pallas_call(
        _gemm_kernel,
        grid_spec=pltpu.PrefetchScalarGridSpec(
            num_scalar_prefetch=1,
            grid=(NTILES,),
            in_specs=[
                pl.BlockSpec(
                    (BT, H),
                    lambda j, te_s: (jnp.minimum(j, te_s[1, 0] - 1), 0)),
                pl.BlockSpec(
                    (BT, 128),
                    lambda j, te_s: (jnp.minimum(j, te_s[1, 0] - 1), 0)),
                pl.BlockSpec(
                    (1, H, F),
                    lambda j, te_s: (te_s[0, jnp.minimum(j, te_s[1, 0] - 1)],
                                     0, 0)),
                pl.BlockSpec(
                    (1, F, H),
                    lambda j, te_s: (te_s[0, jnp.minimum(j, te_s[1, 0] - 1)],
                                     0, 0)),
            ],
            out_specs=pl.BlockSpec(
                (BT, H), lambda j, te_s: (jnp.minimum(j, te_s[1, 0] - 1), 0)),
        ),
        out_shape=jax.ShapeDtypeStruct((NBUF, H), jnp.float32),
    )(teg, x_sorted, g_sorted, experts_w1, experts_w2)

    return _combine(y, pos0, pos1)
